# Initial kernel scaffold; baseline (speedup 1.0000x reference)
import jax
import jax.numpy as jnp
from jax.experimental import pallas as pl


def kernel(x, edge_index, batch, c1_init, c1_w, c1_root, c1_bias, c2_init, c2_w, c2_root, c2_bias, lin_w, lin_b):
    raise NotImplementedError("write your pallas kernel here")


# speedup vs baseline: 186.0998x; 186.0998x over previous
import functools

import jax
import jax.numpy as jnp
from jax import lax
from jax.experimental import pallas as pl
from jax.experimental.pallas import tpu as pltpu
from jax.experimental.pallas import tpu_sc as plsc

N = 50000
NP = 50176
E = 800000
G = 128
NW = 32
NB = 6272
GRID = NP // NB

DEG_S = 25
C1_S = 2
C2_S = 20
B_E = 4000


def _zero_acc(acc):
    z16 = jnp.zeros((16,), jnp.float32)

    @pl.loop(0, NP // 16)
    def _(i):
        acc[pl.ds(i * 16, 16)] = z16


def _make_sc_prop(C, S):
    ES = E // S
    nch = ES // B_E
    assert ES % B_E == 0 and B_E % 16 == 0 and nch % 2 == 0
    units = C * S
    rounds = (units + NW - 1) // NW
    mesh = plsc.VectorSubcoreMesh(core_axis_name="c", subcore_axis_name="s")

    @functools.partial(
        pl.kernel,
        out_type=jax.ShapeDtypeStruct((S, C, NP), jnp.float32),
        mesh=mesh,
        compiler_params=pltpu.CompilerParams(needs_layout_passes=False),
        scratch_types=[
            pltpu.VMEM((NP,), jnp.float32),
            pltpu.VMEM((NP,), jnp.float32),
            pltpu.VMEM((B_E,), jnp.int32),
            pltpu.VMEM((B_E,), jnp.int32),
            pltpu.VMEM((B_E,), jnp.int32),
            pltpu.VMEM((B_E,), jnp.int32),
            pltpu.SemaphoreType.DMA,
            pltpu.SemaphoreType.DMA,
        ],
    )
    def k(tbl_hbm, row_hbm, col_hbm, out_hbm, tbl, acc,
          rbuf0, rbuf1, cbuf0, cbuf1, sem_r, sem_c):
        wid = lax.axis_index("s") * 2 + lax.axis_index("c")
        rbufs = (rbuf0, rbuf1)
        cbufs = (cbuf0, cbuf1)

        def start(parity, ch, ebase):
            off = ebase + ch * B_E
            pltpu.make_async_copy(row_hbm.at[pl.ds(off, B_E)], rbufs[parity], sem_r).start()
            pltpu.make_async_copy(col_hbm.at[pl.ds(off, B_E)], cbufs[parity], sem_c).start()

        def wait(parity):
            pltpu.make_async_copy(row_hbm.at[pl.ds(0, B_E)], rbufs[parity], sem_r).wait()
            pltpu.make_async_copy(col_hbm.at[pl.ds(0, B_E)], cbufs[parity], sem_c).wait()

        def compute(parity):
            @pl.loop(0, B_E // 16)
            def _(i):
                sl = pl.ds(i * 16, 16)
                r16 = rbufs[parity][sl]
                c16 = cbufs[parity][sl]
                vals = plsc.load_gather(tbl, [r16])
                plsc.addupdate_scatter(acc, [c16], vals)

        def do_unit(u):
            c = u % C
            s = u // C
            ebase = s * ES
            pltpu.sync_copy(tbl_hbm.at[c], tbl)
            _zero_acc(acc)
            start(0, 0, ebase)

            @pl.loop(0, nch // 2)
            def _(gp):
                g0 = gp * 2
                wait(0)
                start(1, g0 + 1, ebase)
                compute(0)
                wait(1)

                @pl.when(g0 + 2 < nch)
                def _():
                    start(0, g0 + 2, ebase)

                compute(1)

            pltpu.sync_copy(acc, out_hbm.at[s, c])

        for r in range(rounds):
            u = wid + r * NW
            if (r + 1) * NW <= units:
                do_unit(u)
            else:
                @pl.when(u < units)
                def _():
                    do_unit(u)

    return k


def _make_sc_deg():
    S = DEG_S
    ES = E // S
    nch = ES // B_E
    assert ES % B_E == 0 and nch % 2 == 0
    mesh = plsc.VectorSubcoreMesh(core_axis_name="c", subcore_axis_name="s")

    @functools.partial(
        pl.kernel,
        out_type=jax.ShapeDtypeStruct((S, 1, NP), jnp.float32),
        mesh=mesh,
        compiler_params=pltpu.CompilerParams(needs_layout_passes=False),
        scratch_types=[
            pltpu.VMEM((NP,), jnp.float32),
            pltpu.VMEM((B_E,), jnp.int32),
            pltpu.VMEM((B_E,), jnp.int32),
            pltpu.SemaphoreType.DMA,
        ],
    )
    def k(col_hbm, out_hbm, acc, cbuf0, cbuf1, sem_c):
        wid = lax.axis_index("s") * 2 + lax.axis_index("c")
        ones = jnp.ones((16,), jnp.float32)
        cbufs = (cbuf0, cbuf1)

        def start(parity, ch, ebase):
            off = ebase + ch * B_E
            pltpu.make_async_copy(col_hbm.at[pl.ds(off, B_E)], cbufs[parity], sem_c).start()

        def wait(parity):
            pltpu.make_async_copy(col_hbm.at[pl.ds(0, B_E)], cbufs[parity], sem_c).wait()

        def compute(parity):
            @pl.loop(0, B_E // 16)
            def _(i):
                c16 = cbufs[parity][pl.ds(i * 16, 16)]
                plsc.addupdate_scatter(acc, [c16], ones)

        @pl.when(wid < S)
        def _():
            ebase = wid * ES
            _zero_acc(acc)
            start(0, 0, ebase)

            @pl.loop(0, nch // 2)
            def _(gp):
                g0 = gp * 2
                wait(0)
                start(1, g0 + 1, ebase)
                compute(0)
                wait(1)

                @pl.when(g0 + 2 < nch)
                def _():
                    start(0, g0 + 2, ebase)

                compute(1)

            pltpu.sync_copy(acc, out_hbm.at[wid, 0])

    return k


def _dotT(w, xb):
    return lax.dot_general(w, xb, (((1,), (1,)), ((), ())),
                           preferred_element_type=jnp.float32)


def _dot(w, h):
    return lax.dot_general(w, h, (((1,), (0,)), ((), ())),
                           preferred_element_type=jnp.float32)


def _tc_pre(xp, degp, w1i, w1r, b1):
    def body(x_ref, deg_ref, wi_ref, wr_ref, b_ref, p0_ref, rt_ref, dis_ref):
        xb = x_ref[...]
        deg = deg_ref[...].sum(axis=0)
        dis = jnp.where(deg > 0, lax.rsqrt(deg), 0.0)
        p0_ref[...] = _dotT(wi_ref[...], xb) * dis
        rt_ref[...] = _dotT(wr_ref[...], xb) + b_ref[...]
        dis_ref[...] = dis

    return pl.pallas_call(
        body,
        grid=(GRID,),
        in_specs=[
            pl.BlockSpec((NB, 75), lambda i: (i, 0)),
            pl.BlockSpec((DEG_S, 1, NB), lambda i: (0, 0, i)),
            pl.BlockSpec((48, 75), lambda i: (0, 0)),
            pl.BlockSpec((48, 75), lambda i: (0, 0)),
            pl.BlockSpec((48, 1), lambda i: (0, 0)),
        ],
        out_specs=[
            pl.BlockSpec((48, NB), lambda i: (0, i)),
            pl.BlockSpec((48, NB), lambda i: (0, i)),
            pl.BlockSpec((1, NB), lambda i: (0, i)),
        ],
        out_shape=[
            jax.ShapeDtypeStruct((48, NP), jnp.float32),
            jax.ShapeDtypeStruct((48, NP), jnp.float32),
            jax.ShapeDtypeStruct((1, NP), jnp.float32),
        ],
    )(xp, degp, w1i, w1r, b1)


def _tc_mid(u0p, rt, dis, w1wt):
    def body(up_ref, rt_ref, dis_ref, w_ref, q_ref):
        u0 = up_ref[...].sum(axis=0)
        dis = dis_ref[...]
        ut = jnp.maximum(u0 * dis + rt_ref[...], 0.0)
        w = w_ref[...]
        q = jnp.concatenate(
            [_dot(w[k], ut[k * 16:(k + 1) * 16, :]) for k in range(3)], axis=0)
        q_ref[...] = q * dis

    return pl.pallas_call(
        body,
        grid=(GRID,),
        in_specs=[
            pl.BlockSpec((C1_S, 48, NB), lambda i: (0, 0, i)),
            pl.BlockSpec((48, NB), lambda i: (0, i)),
            pl.BlockSpec((1, NB), lambda i: (0, i)),
            pl.BlockSpec((3, 16, 16), lambda i: (0, 0, 0)),
        ],
        out_specs=pl.BlockSpec((48, NB), lambda i: (0, i)),
        out_shape=jax.ShapeDtypeStruct((48, NP), jnp.float32),
    )(u0p, rt, dis, w1wt)


def _tc_h1(v0p, rt, dis, ma):
    def body(vp_ref, rt_ref, dis_ref, ma_ref, s0_ref, h1_ref):
        v0 = vp_ref[...].sum(axis=0)
        dis = dis_ref[...]
        vt = jnp.maximum(v0 * dis + rt_ref[...], 0.0)
        h1 = jnp.maximum((vt[0:16] + vt[16:32] + vt[32:48]) * (1.0 / 3.0), 0.0)
        s0_ref[...] = _dot(ma_ref[...], h1) * dis
        h1_ref[...] = h1

    return pl.pallas_call(
        body,
        grid=(GRID,),
        in_specs=[
            pl.BlockSpec((C1_S, 48, NB), lambda i: (0, 0, i)),
            pl.BlockSpec((48, NB), lambda i: (0, i)),
            pl.BlockSpec((1, NB), lambda i: (0, i)),
            pl.BlockSpec((3, 16), lambda i: (0, 0)),
        ],
        out_specs=[
            pl.BlockSpec((3, NB), lambda i: (0, i)),
            pl.BlockSpec((16, NB), lambda i: (0, i)),
        ],
        out_shape=[
            jax.ShapeDtypeStruct((3, NP), jnp.float32),
            jax.ShapeDtypeStruct((16, NP), jnp.float32),
        ],
    )(v0p, rt, dis, ma)


def _tc_m(w0p, h1, dis, mb, beta):
    def body(wp_ref, h1_ref, dis_ref, mb_ref, beta_ref, ms_ref):
        w0 = wp_ref[...].sum(axis=0)
        dis = dis_ref[...]
        m = w0 * dis + _dot(mb_ref[...], h1_ref[...]) + beta_ref[...]
        ms_ref[...] = m * dis

    return pl.pallas_call(
        body,
        grid=(GRID,),
        in_specs=[
            pl.BlockSpec((C2_S, 3, NB), lambda i: (0, 0, i)),
            pl.BlockSpec((16, NB), lambda i: (0, i)),
            pl.BlockSpec((1, NB), lambda i: (0, i)),
            pl.BlockSpec((3, 16), lambda i: (0, 0)),
            pl.BlockSpec((3, 1), lambda i: (0, 0)),
        ],
        out_specs=pl.BlockSpec((3, NB), lambda i: (0, i)),
        out_shape=jax.ShapeDtypeStruct((3, NP), jnp.float32),
    )(w0p, h1, dis, mb, beta)


def _tc_out(w1p, h1, dis, mc, gamma, batch_p, lin_b):
    def body(wp_ref, h1_ref, dis_ref, mc_ref, gam_ref, b_ref, linb_ref, z_ref):
        i = pl.program_id(0)
        w1 = wp_ref[...].sum(axis=0)
        dis = dis_ref[...]
        sig3 = w1 * dis + _dot(mc_ref[...], h1_ref[...]) + gam_ref[...]
        sig = sig3.sum(axis=0, keepdims=True) * (1.0 / 3.0)
        oh = (b_ref[...] == lax.broadcasted_iota(jnp.int32, (G, NB), 0))
        zc = lax.dot_general(oh.astype(jnp.float32), sig,
                             (((1,), (1,)), ((), ())),
                             preferred_element_type=jnp.float32)

        @pl.when(i == 0)
        def _():
            z_ref[...] = linb_ref[...] + zc

        @pl.when(i != 0)
        def _():
            z_ref[...] += zc

    return pl.pallas_call(
        body,
        grid=(GRID,),
        in_specs=[
            pl.BlockSpec((C2_S, 3, NB), lambda i: (0, 0, i)),
            pl.BlockSpec((16, NB), lambda i: (0, i)),
            pl.BlockSpec((1, NB), lambda i: (0, i)),
            pl.BlockSpec((3, 16), lambda i: (0, 0)),
            pl.BlockSpec((3, 1), lambda i: (0, 0)),
            pl.BlockSpec((1, NB), lambda i: (0, i)),
            pl.BlockSpec((1, 1), lambda i: (0, 0)),
        ],
        out_specs=pl.BlockSpec((G, 1), lambda i: (0, 0)),
        out_shape=jax.ShapeDtypeStruct((G, 1), jnp.float32),
    )(w1p, h1, dis, mc, gamma, batch_p, lin_b)


_sc_deg = _make_sc_deg()
_sc_prop48 = _make_sc_prop(48, C1_S)
_sc_prop3 = _make_sc_prop(3, C2_S)


def kernel(x, edge_index, batch, c1_init, c1_w, c1_root, c1_bias,
           c2_init, c2_w, c2_root, c2_bias, lin_w, lin_b):
    row = edge_index[0]
    col = edge_index[1]

    xp = jnp.pad(x, ((0, NP - N), (0, 0)))
    batch_p = jnp.pad(batch, (0, NP - N), constant_values=jnp.int32(2 ** 30))
    batch_p = batch_p.reshape(1, NP)
    w1i = c1_init.transpose(0, 2, 1).reshape(48, 75)
    w1r = c1_root[0].transpose(0, 2, 1).reshape(48, 75)
    b1 = c1_bias[0].reshape(48, 1)
    w1wt = c1_w[0].transpose(0, 2, 1)
    p = lin_w[0]
    c2w0, c2r0, c2b0 = c2_w[0], c2_root[0], c2_bias[0][:, 0, :]
    q = jnp.einsum('kij,j->ki', c2w0, p)
    ma = jnp.einsum('kfj,kj->kf', c2_init, q)
    mb = jnp.einsum('kfj,kj->kf', c2r0, q)
    beta = jnp.einsum('kj,kj->k', c2b0, q).reshape(3, 1)
    mc = jnp.einsum('kfj,j->kf', c2r0, p)
    gamma = (c2b0 @ p).reshape(3, 1)
    lin_b2 = lin_b.reshape(1, 1)

    degp = _sc_deg(col)
    p0s, rt, dis = _tc_pre(xp, degp, w1i, w1r, b1)
    u0p = _sc_prop48(p0s, row, col)
    qs = _tc_mid(u0p, rt, dis, w1wt)
    v0p = _sc_prop48(qs, row, col)
    s0s, h1 = _tc_h1(v0p, rt, dis, ma)
    w0p = _sc_prop3(s0s, row, col)
    ms = _tc_m(w0p, h1, dis, mb, beta)
    w1p = _sc_prop3(ms, row, col)
    z = _tc_out(w1p, h1, dis, mc, gamma, batch_p, lin_b2)
    return z


# speedup vs baseline: 436.7971x; 2.3471x over previous
import functools

import jax
import jax.numpy as jnp
from jax import lax
from jax.experimental import pallas as pl
from jax.experimental.pallas import tpu as pltpu
from jax.experimental.pallas import tpu_sc as plsc

N = 50000
NP = 50176
E = 800000
G = 128
NW = 32
NB = 6272
GRID = NP // NB

DEG_S = 25
C1_S = 2
C2_S = 20
B_E = 4000


def _zero_acc(acc):
    z16 = jnp.zeros((16,), jnp.float32)

    @plsc.parallel_loop(0, NP // 16, unroll=8)
    def _(i):
        acc[pl.ds(i * 16, 16)] = z16


def _make_sc_prop(C, S):
    ES = E // S
    nch = ES // B_E
    assert ES % B_E == 0 and B_E % 16 == 0 and nch % 2 == 0
    units = C * S
    rounds = (units + NW - 1) // NW
    mesh = plsc.VectorSubcoreMesh(core_axis_name="c", subcore_axis_name="s")

    @functools.partial(
        pl.kernel,
        out_type=jax.ShapeDtypeStruct((S, C, NP), jnp.float32),
        mesh=mesh,
        compiler_params=pltpu.CompilerParams(needs_layout_passes=False),
        scratch_types=[
            pltpu.VMEM((NP,), jnp.float32),
            pltpu.VMEM((NP,), jnp.float32),
            pltpu.VMEM((B_E,), jnp.int32),
            pltpu.VMEM((B_E,), jnp.int32),
            pltpu.VMEM((B_E,), jnp.int32),
            pltpu.VMEM((B_E,), jnp.int32),
            pltpu.SemaphoreType.DMA,
            pltpu.SemaphoreType.DMA,
        ],
    )
    def k(tbl_hbm, ei_hbm, out_hbm, tbl, acc,
          rbuf0, rbuf1, cbuf0, cbuf1, sem_r, sem_c):
        wid = lax.axis_index("s") * 2 + lax.axis_index("c")
        rbufs = (rbuf0, rbuf1)
        cbufs = (cbuf0, cbuf1)

        def start(parity, ch, ebase):
            off = ebase + ch * B_E
            pltpu.make_async_copy(ei_hbm.at[pl.ds(off, B_E)], rbufs[parity], sem_r).start()
            pltpu.make_async_copy(ei_hbm.at[pl.ds(E + off, B_E)], cbufs[parity], sem_c).start()

        def wait(parity):
            pltpu.make_async_copy(ei_hbm.at[pl.ds(0, B_E)], rbufs[parity], sem_r).wait()
            pltpu.make_async_copy(ei_hbm.at[pl.ds(0, B_E)], cbufs[parity], sem_c).wait()

        def compute(parity):
            @plsc.parallel_loop(0, B_E // 16, unroll=8)
            def _(i):
                sl = pl.ds(i * 16, 16)
                r16 = rbufs[parity][sl]
                c16 = cbufs[parity][sl]
                vals = plsc.load_gather(tbl, [r16])
                plsc.addupdate_scatter(acc, [c16], vals)

        def do_unit(u):
            c = u % C
            s = u // C
            ebase = s * ES
            pltpu.sync_copy(tbl_hbm.at[c], tbl)
            _zero_acc(acc)
            start(0, 0, ebase)

            @pl.loop(0, nch // 2)
            def _(gp):
                g0 = gp * 2
                wait(0)
                start(1, g0 + 1, ebase)
                compute(0)
                wait(1)

                @pl.when(g0 + 2 < nch)
                def _():
                    start(0, g0 + 2, ebase)

                compute(1)

            pltpu.sync_copy(acc, out_hbm.at[s, c])

        for r in range(rounds):
            u = wid + r * NW
            if (r + 1) * NW <= units:
                do_unit(u)
            else:
                @pl.when(u < units)
                def _():
                    do_unit(u)

    return k


def _make_sc_deg():
    S = DEG_S
    ES = E // S
    nch = ES // B_E
    assert ES % B_E == 0 and nch % 2 == 0
    mesh = plsc.VectorSubcoreMesh(core_axis_name="c", subcore_axis_name="s")

    @functools.partial(
        pl.kernel,
        out_type=jax.ShapeDtypeStruct((S, 1, NP), jnp.float32),
        mesh=mesh,
        compiler_params=pltpu.CompilerParams(needs_layout_passes=False),
        scratch_types=[
            pltpu.VMEM((NP,), jnp.float32),
            pltpu.VMEM((B_E,), jnp.int32),
            pltpu.VMEM((B_E,), jnp.int32),
            pltpu.SemaphoreType.DMA,
        ],
    )
    def k(ei_hbm, out_hbm, acc, cbuf0, cbuf1, sem_c):
        wid = lax.axis_index("s") * 2 + lax.axis_index("c")
        ones = jnp.ones((16,), jnp.float32)
        cbufs = (cbuf0, cbuf1)

        def start(parity, ch, ebase):
            off = ebase + ch * B_E
            pltpu.make_async_copy(ei_hbm.at[pl.ds(E + off, B_E)], cbufs[parity], sem_c).start()

        def wait(parity):
            pltpu.make_async_copy(ei_hbm.at[pl.ds(0, B_E)], cbufs[parity], sem_c).wait()

        def compute(parity):
            @plsc.parallel_loop(0, B_E // 16, unroll=8)
            def _(i):
                c16 = cbufs[parity][pl.ds(i * 16, 16)]
                plsc.addupdate_scatter(acc, [c16], ones)

        @pl.when(wid < S)
        def _():
            ebase = wid * ES
            _zero_acc(acc)
            start(0, 0, ebase)

            @pl.loop(0, nch // 2)
            def _(gp):
                g0 = gp * 2
                wait(0)
                start(1, g0 + 1, ebase)
                compute(0)
                wait(1)

                @pl.when(g0 + 2 < nch)
                def _():
                    start(0, g0 + 2, ebase)

                compute(1)

            pltpu.sync_copy(acc, out_hbm.at[wid, 0])

    return k


def _dotT(w, xb):
    return lax.dot_general(w, xb, (((1,), (1,)), ((), ())),
                           preferred_element_type=jnp.float32)


def _dot(w, h):
    return lax.dot_general(w, h, (((1,), (0,)), ((), ())),
                           preferred_element_type=jnp.float32)


def _tc_pre(xp, degp, w1i, w1r, b1):
    def body(x_ref, deg_ref, wi_ref, wr_ref, b_ref, p0_ref, rt_ref, dis_ref):
        xb = x_ref[...]
        deg = deg_ref[...].sum(axis=0)
        dis = jnp.where(deg > 0, lax.rsqrt(deg), 0.0)
        p0_ref[...] = _dotT(wi_ref[...], xb) * dis
        rt_ref[...] = _dotT(wr_ref[...], xb) + b_ref[...]
        dis_ref[...] = dis

    return pl.pallas_call(
        body,
        grid=(GRID,),
        in_specs=[
            pl.BlockSpec((NB, 75), lambda i: (i, 0)),
            pl.BlockSpec((DEG_S, 1, NB), lambda i: (0, 0, i)),
            pl.BlockSpec((48, 75), lambda i: (0, 0)),
            pl.BlockSpec((48, 75), lambda i: (0, 0)),
            pl.BlockSpec((48, 1), lambda i: (0, 0)),
        ],
        out_specs=[
            pl.BlockSpec((48, NB), lambda i: (0, i)),
            pl.BlockSpec((48, NB), lambda i: (0, i)),
            pl.BlockSpec((1, NB), lambda i: (0, i)),
        ],
        out_shape=[
            jax.ShapeDtypeStruct((48, NP), jnp.float32),
            jax.ShapeDtypeStruct((48, NP), jnp.float32),
            jax.ShapeDtypeStruct((1, NP), jnp.float32),
        ],
    )(xp, degp, w1i, w1r, b1)


def _tc_mid(u0p, rt, dis, w1wt):
    def body(up_ref, rt_ref, dis_ref, w_ref, q_ref):
        u0 = up_ref[...].sum(axis=0)
        dis = dis_ref[...]
        ut = jnp.maximum(u0 * dis + rt_ref[...], 0.0)
        w = w_ref[...]
        q = jnp.concatenate(
            [_dot(w[k], ut[k * 16:(k + 1) * 16, :]) for k in range(3)], axis=0)
        q_ref[...] = q * dis

    return pl.pallas_call(
        body,
        grid=(GRID,),
        in_specs=[
            pl.BlockSpec((C1_S, 48, NB), lambda i: (0, 0, i)),
            pl.BlockSpec((48, NB), lambda i: (0, i)),
            pl.BlockSpec((1, NB), lambda i: (0, i)),
            pl.BlockSpec((3, 16, 16), lambda i: (0, 0, 0)),
        ],
        out_specs=pl.BlockSpec((48, NB), lambda i: (0, i)),
        out_shape=jax.ShapeDtypeStruct((48, NP), jnp.float32),
    )(u0p, rt, dis, w1wt)


def _tc_h1(v0p, rt, dis, ma):
    def body(vp_ref, rt_ref, dis_ref, ma_ref, s0_ref, h1_ref):
        v0 = vp_ref[...].sum(axis=0)
        dis = dis_ref[...]
        vt = jnp.maximum(v0 * dis + rt_ref[...], 0.0)
        h1 = jnp.maximum((vt[0:16] + vt[16:32] + vt[32:48]) * (1.0 / 3.0), 0.0)
        s0_ref[...] = _dot(ma_ref[...], h1) * dis
        h1_ref[...] = h1

    return pl.pallas_call(
        body,
        grid=(GRID,),
        in_specs=[
            pl.BlockSpec((C1_S, 48, NB), lambda i: (0, 0, i)),
            pl.BlockSpec((48, NB), lambda i: (0, i)),
            pl.BlockSpec((1, NB), lambda i: (0, i)),
            pl.BlockSpec((3, 16), lambda i: (0, 0)),
        ],
        out_specs=[
            pl.BlockSpec((3, NB), lambda i: (0, i)),
            pl.BlockSpec((16, NB), lambda i: (0, i)),
        ],
        out_shape=[
            jax.ShapeDtypeStruct((3, NP), jnp.float32),
            jax.ShapeDtypeStruct((16, NP), jnp.float32),
        ],
    )(v0p, rt, dis, ma)


def _tc_m(w0p, h1, dis, mb, beta):
    def body(wp_ref, h1_ref, dis_ref, mb_ref, beta_ref, ms_ref):
        w0 = wp_ref[...].sum(axis=0)
        dis = dis_ref[...]
        m = w0 * dis + _dot(mb_ref[...], h1_ref[...]) + beta_ref[...]
        ms_ref[...] = m * dis

    return pl.pallas_call(
        body,
        grid=(GRID,),
        in_specs=[
            pl.BlockSpec((C2_S, 3, NB), lambda i: (0, 0, i)),
            pl.BlockSpec((16, NB), lambda i: (0, i)),
            pl.BlockSpec((1, NB), lambda i: (0, i)),
            pl.BlockSpec((3, 16), lambda i: (0, 0)),
            pl.BlockSpec((3, 1), lambda i: (0, 0)),
        ],
        out_specs=pl.BlockSpec((3, NB), lambda i: (0, i)),
        out_shape=jax.ShapeDtypeStruct((3, NP), jnp.float32),
    )(w0p, h1, dis, mb, beta)


def _tc_out(w1p, h1, dis, mc, gamma, batch_p, lin_b):
    def body(wp_ref, h1_ref, dis_ref, mc_ref, gam_ref, b_ref, linb_ref, z_ref):
        i = pl.program_id(0)
        w1 = wp_ref[...].sum(axis=0)
        dis = dis_ref[...]
        sig3 = w1 * dis + _dot(mc_ref[...], h1_ref[...]) + gam_ref[...]
        sig = sig3.sum(axis=0, keepdims=True) * (1.0 / 3.0)
        oh = (b_ref[...] == lax.broadcasted_iota(jnp.int32, (G, NB), 0))
        zc = lax.dot_general(oh.astype(jnp.float32), sig,
                             (((1,), (1,)), ((), ())),
                             preferred_element_type=jnp.float32)

        @pl.when(i == 0)
        def _():
            z_ref[...] = linb_ref[...] + zc

        @pl.when(i != 0)
        def _():
            z_ref[...] += zc

    return pl.pallas_call(
        body,
        grid=(GRID,),
        in_specs=[
            pl.BlockSpec((C2_S, 3, NB), lambda i: (0, 0, i)),
            pl.BlockSpec((16, NB), lambda i: (0, i)),
            pl.BlockSpec((1, NB), lambda i: (0, i)),
            pl.BlockSpec((3, 16), lambda i: (0, 0)),
            pl.BlockSpec((3, 1), lambda i: (0, 0)),
            pl.BlockSpec((1, NB), lambda i: (0, i)),
            pl.BlockSpec((1, 1), lambda i: (0, 0)),
        ],
        out_specs=pl.BlockSpec((G, 1), lambda i: (0, 0)),
        out_shape=jax.ShapeDtypeStruct((G, 1), jnp.float32),
    )(w1p, h1, dis, mc, gamma, batch_p, lin_b)


_sc_deg = _make_sc_deg()
_sc_prop48 = _make_sc_prop(48, C1_S)
_sc_prop3 = _make_sc_prop(3, C2_S)


def kernel(x, edge_index, batch, c1_init, c1_w, c1_root, c1_bias,
           c2_init, c2_w, c2_root, c2_bias, lin_w, lin_b):
    ei_flat = edge_index.reshape(2 * E)

    xp = jnp.pad(x, ((0, NP - N), (0, 0)))
    batch_p = jnp.pad(batch, (0, NP - N), constant_values=jnp.int32(2 ** 30))
    batch_p = batch_p.reshape(1, NP)
    w1i = c1_init.transpose(0, 2, 1).reshape(48, 75)
    w1r = c1_root[0].transpose(0, 2, 1).reshape(48, 75)
    b1 = c1_bias[0].reshape(48, 1)
    w1wt = c1_w[0].transpose(0, 2, 1)
    p = lin_w[0]
    c2w0, c2r0, c2b0 = c2_w[0], c2_root[0], c2_bias[0][:, 0, :]
    q = jnp.einsum('kij,j->ki', c2w0, p)
    ma = jnp.einsum('kfj,kj->kf', c2_init, q)
    mb = jnp.einsum('kfj,kj->kf', c2r0, q)
    beta = jnp.einsum('kj,kj->k', c2b0, q).reshape(3, 1)
    mc = jnp.einsum('kfj,j->kf', c2r0, p)
    gamma = (c2b0 @ p).reshape(3, 1)
    lin_b2 = lin_b.reshape(1, 1)

    degp = _sc_deg(ei_flat)
    p0s, rt, dis = _tc_pre(xp, degp, w1i, w1r, b1)
    u0p = _sc_prop48(p0s, ei_flat)
    qs = _tc_mid(u0p, rt, dis, w1wt)
    v0p = _sc_prop48(qs, ei_flat)
    s0s, h1 = _tc_h1(v0p, rt, dis, ma)
    w0p = _sc_prop3(s0s, ei_flat)
    ms = _tc_m(w0p, h1, dis, mb, beta)
    w1p = _sc_prop3(ms, ei_flat)
    z = _tc_out(w1p, h1, dis, mc, gamma, batch_p, lin_b2)
    return z
